# initial kernel scaffold (unmeasured)
import jax
import jax.numpy as jnp
from jax import lax
from jax.experimental import pallas as pl
from jax.experimental.pallas import tpu as pltpu

N_DEV = 16
N_LOC = 2
CAP = 64
SLAB = N_LOC * CAP


def _moe_body(send_ref, w1_ref, w2_ref, ret_ref, recv_ref, res_ref,
              send_sems, recv_sems, send_sems2, recv_sems2):
    me = lax.axis_index("i")

    bsem = pltpu.get_barrier_semaphore()
    for k in range(N_DEV):
        pl.semaphore_signal(bsem, inc=1, device_id=(k,),
                            device_id_type=pl.DeviceIdType.MESH)
    pl.semaphore_wait(bsem, N_DEV)

    dispatch = []
    for k in range(N_DEV):
        rdma = pltpu.make_async_remote_copy(
            src_ref=send_ref.at[k],
            dst_ref=recv_ref.at[me],
            send_sem=send_sems.at[k],
            recv_sem=recv_sems.at[k],
            device_id=(k,),
            device_id_type=pl.DeviceIdType.MESH,
        )
        rdma.start()
        dispatch.append(rdma)

    for s in range(N_DEV):
        pltpu.make_async_remote_copy(
            src_ref=send_ref.at[s],
            dst_ref=recv_ref.at[s],
            send_sem=send_sems.at[s],
            recv_sem=recv_sems.at[s],
            device_id=(s,),
            device_id_type=pl.DeviceIdType.MESH,
        ).wait_recv()

    d_model = send_ref.shape[-1]
    for le in range(N_LOC):
        a = recv_ref[:, le].reshape(N_DEV * CAP, d_model)
        h = jnp.maximum(
            jnp.dot(a, w1_ref[le], preferred_element_type=jnp.float32), 0.0)
        r = jnp.dot(h, w2_ref[le], preferred_element_type=jnp.float32)
        res_ref[:, le] = r.reshape(N_DEV, CAP, d_model)

    combine = []
    for k in range(N_DEV):
        rdma = pltpu.make_async_remote_copy(
            src_ref=res_ref.at[k],
            dst_ref=ret_ref.at[me],
            send_sem=send_sems2.at[k],
            recv_sem=recv_sems2.at[k],
            device_id=(k,),
            device_id_type=pl.DeviceIdType.MESH,
        )
        rdma.start()
        combine.append(rdma)

    for s in range(N_DEV):
        pltpu.make_async_remote_copy(
            src_ref=res_ref.at[s],
            dst_ref=ret_ref.at[s],
            send_sem=send_sems2.at[s],
            recv_sem=recv_sems2.at[s],
            device_id=(s,),
            device_id_type=pl.DeviceIdType.MESH,
        ).wait_recv()

    for rdma in dispatch:
        rdma.wait_send()
    for rdma in combine:
        rdma.wait_send()


def kernel(x, assign, W1, W2):
    t_per, d_model = x.shape

    a = assign.astype(jnp.int32)
    tok = jnp.arange(t_per, dtype=jnp.int32)
    same = (a[:, None] == a[None, :]) & (tok[None, :] < tok[:, None])
    rank = jnp.sum(same, axis=1).astype(jnp.int32)
    slot = (a // N_LOC) * SLAB + (a % N_LOC) * CAP + rank

    token_of_slot = jnp.zeros((N_DEV * SLAB,), jnp.int32).at[slot].set(tok)
    send = x[token_of_slot].reshape(N_DEV, N_LOC, CAP, d_model)

    f_hidden = W1.shape[-1]
    ret = pl.pallas_call(
        _moe_body,
        out_shape=jax.ShapeDtypeStruct((N_DEV, N_LOC, CAP, d_model),
                                       jnp.float32),
        in_specs=[
            pl.BlockSpec(memory_space=pltpu.VMEM),
            pl.BlockSpec(memory_space=pltpu.VMEM),
            pl.BlockSpec(memory_space=pltpu.VMEM),
        ],
        out_specs=pl.BlockSpec(memory_space=pltpu.VMEM),
        scratch_shapes=[
            pltpu.VMEM((N_DEV, N_LOC, CAP, d_model), jnp.float32),
            pltpu.VMEM((N_DEV, N_LOC, CAP, d_model), jnp.float32),
            pltpu.SemaphoreType.DMA((N_DEV,)),
            pltpu.SemaphoreType.DMA((N_DEV,)),
            pltpu.SemaphoreType.DMA((N_DEV,)),
            pltpu.SemaphoreType.DMA((N_DEV,)),
        ],
        compiler_params=pltpu.CompilerParams(collective_id=0),
    )(send, W1, W2)

    return ret.reshape(N_DEV * SLAB, d_model)[slot]


# baseline (device time: 390446 ns/iter reference)
import jax
import jax.numpy as jnp
from jax import lax
from jax.experimental import pallas as pl
from jax.experimental.pallas import tpu as pltpu

N_DEV = 16
N_LOC = 2
CAP = 64
SLAB = N_LOC * CAP


def _moe_body(send_ref, w1_ref, w2_ref, ret_ref, recv_ref,
              send_sems, recv_sems, send_sems2, recv_sems2):
    me = lax.axis_index("i")

    bsem = pltpu.get_barrier_semaphore()
    for k in range(N_DEV):
        pl.semaphore_signal(bsem, inc=1, device_id=(k,),
                            device_id_type=pl.DeviceIdType.MESH)
    pl.semaphore_wait(bsem, N_DEV)

    recv_ref[pl.ds(me, 1)] = send_ref[pl.ds(me, 1)]
    for r in range(1, N_DEV):
        dst = lax.rem(me + r, N_DEV)
        src = lax.rem(me + N_DEV - r, N_DEV)
        rdma = pltpu.make_async_remote_copy(
            src_ref=send_ref.at[dst],
            dst_ref=recv_ref.at[me],
            send_sem=send_sems.at[r],
            recv_sem=recv_sems.at[me],
            device_id=(dst,),
            device_id_type=pl.DeviceIdType.MESH,
        )
        rdma.start()
        pltpu.make_async_remote_copy(
            src_ref=send_ref.at[src],
            dst_ref=recv_ref.at[src],
            send_sem=send_sems.at[r],
            recv_sem=recv_sems.at[src],
            device_id=(src,),
            device_id_type=pl.DeviceIdType.MESH,
        ).wait_recv()
        rdma.wait_send()

    d_model = send_ref.shape[-1]
    TILE = 256
    rows_per_exp = N_DEV * CAP
    dev_per_tile = TILE // CAP
    for le in range(N_LOC):
        for t0 in range(0, rows_per_exp // TILE):
            dlo = t0 * dev_per_tile
            a = recv_ref[dlo:dlo + dev_per_tile, le].reshape(TILE, d_model)
            h = jnp.maximum(
                jnp.dot(a, w1_ref[le], preferred_element_type=jnp.float32),
                0.0)
            r = jnp.dot(h, w2_ref[le], preferred_element_type=jnp.float32)
            recv_ref[dlo:dlo + dev_per_tile, le] = r.reshape(
                dev_per_tile, CAP, d_model)

    ret_ref[pl.ds(me, 1)] = recv_ref[pl.ds(me, 1)]
    for r in range(1, N_DEV):
        dst = lax.rem(me + r, N_DEV)
        src = lax.rem(me + N_DEV - r, N_DEV)
        rdma = pltpu.make_async_remote_copy(
            src_ref=recv_ref.at[dst],
            dst_ref=ret_ref.at[me],
            send_sem=send_sems2.at[r],
            recv_sem=recv_sems2.at[me],
            device_id=(dst,),
            device_id_type=pl.DeviceIdType.MESH,
        )
        rdma.start()
        pltpu.make_async_remote_copy(
            src_ref=recv_ref.at[src],
            dst_ref=ret_ref.at[src],
            send_sem=send_sems2.at[r],
            recv_sem=recv_sems2.at[src],
            device_id=(src,),
            device_id_type=pl.DeviceIdType.MESH,
        ).wait_recv()
        rdma.wait_send()


def kernel(x, assign, W1, W2):
    t_per, d_model = x.shape

    a = assign.astype(jnp.int32)
    tok = jnp.arange(t_per, dtype=jnp.int32)
    same = (a[:, None] == a[None, :]) & (tok[None, :] < tok[:, None])
    rank = jnp.sum(same, axis=1).astype(jnp.int32)
    slot = (a // N_LOC) * SLAB + (a % N_LOC) * CAP + rank

    token_of_slot = jnp.zeros((N_DEV * SLAB,), jnp.int32).at[slot].set(tok)
    send = x[token_of_slot].reshape(N_DEV, N_LOC, CAP, d_model)

    f_hidden = W1.shape[-1]
    ret = pl.pallas_call(
        _moe_body,
        out_shape=jax.ShapeDtypeStruct((N_DEV, N_LOC, CAP, d_model),
                                       jnp.float32),
        in_specs=[
            pl.BlockSpec(memory_space=pltpu.VMEM),
            pl.BlockSpec(memory_space=pltpu.VMEM),
            pl.BlockSpec(memory_space=pltpu.VMEM),
        ],
        out_specs=pl.BlockSpec(memory_space=pltpu.VMEM),
        scratch_shapes=[
            pltpu.VMEM((N_DEV, N_LOC, CAP, d_model), jnp.float32),
            pltpu.SemaphoreType.DMA((N_DEV,)),
            pltpu.SemaphoreType.DMA((N_DEV,)),
            pltpu.SemaphoreType.DMA((N_DEV,)),
            pltpu.SemaphoreType.DMA((N_DEV,)),
        ],
        compiler_params=pltpu.CompilerParams(
            collective_id=0, vmem_limit_bytes=120 * 1024 * 1024),
    )(send, W1, W2)

    return ret.reshape(N_DEV * SLAB, d_model)[slot]


# device time: 296035 ns/iter; 1.3189x vs baseline; 1.3189x over previous
import jax
import jax.numpy as jnp
from jax import lax
from jax.experimental import pallas as pl
from jax.experimental.pallas import tpu as pltpu

N_DEV = 16
N_LOC = 2
CAP = 64
SLAB = N_LOC * CAP
G = 4
W = 4


def _moe_body(send_ref, w1_ref, w2_ref, ret_ref, recv_ref,
              send_sems, recv_sems, send_sems2, recv_sems2):
    me = lax.axis_index("i")

    bsem = pltpu.get_barrier_semaphore()
    for k in range(N_DEV):
        pl.when(me != k)(
            lambda k=k: pl.semaphore_signal(
                bsem, inc=1, device_id=(k,),
                device_id_type=pl.DeviceIdType.MESH))
    pl.semaphore_wait(bsem, N_DEV - 1)

    d_model = send_ref.shape[-1]

    def disp(r):
        return pltpu.make_async_remote_copy(
            src_ref=send_ref.at[r],
            dst_ref=recv_ref.at[r],
            send_sem=send_sems.at[r],
            recv_sem=recv_sems.at[r],
            device_id=(lax.rem(me + r, N_DEV),),
            device_id_type=pl.DeviceIdType.MESH,
        )

    def comb(r):
        return pltpu.make_async_remote_copy(
            src_ref=recv_ref.at[r],
            dst_ref=ret_ref.at[r],
            send_sem=send_sems2.at[r],
            recv_sem=recv_sems2.at[r],
            device_id=(lax.rem(me + N_DEV - r, N_DEV),),
            device_id_type=pl.DeviceIdType.MESH,
        )

    recv_ref[0] = send_ref[0]
    dispatch = {r: disp(r) for r in range(1, N_DEV)}
    combine = {r: comb(r) for r in range(1, N_DEV)}

    for r in range(1, min(W, N_DEV - 1) + 1):
        dispatch[r].start()

    n_tiles = N_DEV // G
    for j in range(n_tiles):
        for r in range(j * G, (j + 1) * G):
            if r >= 1 and r + W < N_DEV:
                dispatch[r + W].start()
            if r >= 1:
                dispatch[r].wait_recv()

        lo = j * G
        for le in range(N_LOC):
            a = recv_ref[lo:lo + G, le].reshape(G * CAP, d_model)
            h = jnp.maximum(
                jnp.dot(a, w1_ref[le], preferred_element_type=jnp.float32),
                0.0)
            res = jnp.dot(h, w2_ref[le], preferred_element_type=jnp.float32)
            recv_ref[lo:lo + G, le] = res.reshape(G, CAP, d_model)

        for r in range(lo, lo + G):
            if r == 0:
                ret_ref[0] = recv_ref[0]
            else:
                combine[r].start()

    for r in range(1, N_DEV):
        combine[r].wait_recv()
        combine[r].wait_send()
        dispatch[r].wait_send()


def kernel(x, assign, W1, W2):
    t_per, d_model = x.shape
    me = lax.axis_index("i")

    a = assign.astype(jnp.int32)
    tok = jnp.arange(t_per, dtype=jnp.int32)
    same = (a[:, None] == a[None, :]) & (tok[None, :] < tok[:, None])
    rank = jnp.sum(same, axis=1).astype(jnp.int32)
    owner = a // N_LOC
    rnd = jnp.remainder(owner - me, N_DEV)
    slot = rnd * SLAB + (a % N_LOC) * CAP + rank

    token_of_slot = jnp.zeros((N_DEV * SLAB,), jnp.int32).at[slot].set(tok)
    send = x[token_of_slot].reshape(N_DEV, N_LOC, CAP, d_model)

    ret = pl.pallas_call(
        _moe_body,
        out_shape=jax.ShapeDtypeStruct((N_DEV, N_LOC, CAP, d_model),
                                       jnp.float32),
        in_specs=[
            pl.BlockSpec(memory_space=pltpu.VMEM),
            pl.BlockSpec(memory_space=pltpu.VMEM),
            pl.BlockSpec(memory_space=pltpu.VMEM),
        ],
        out_specs=pl.BlockSpec(memory_space=pltpu.VMEM),
        scratch_shapes=[
            pltpu.VMEM((N_DEV, N_LOC, CAP, d_model), jnp.float32),
            pltpu.SemaphoreType.DMA((N_DEV,)),
            pltpu.SemaphoreType.DMA((N_DEV,)),
            pltpu.SemaphoreType.DMA((N_DEV,)),
            pltpu.SemaphoreType.DMA((N_DEV,)),
        ],
        compiler_params=pltpu.CompilerParams(
            collective_id=0, vmem_limit_bytes=120 * 1024 * 1024),
    )(send, W1, W2)

    return ret.reshape(N_DEV * SLAB, d_model)[slot]


# device time: 168354 ns/iter; 2.3192x vs baseline; 1.7584x over previous
import jax
import jax.numpy as jnp
from jax import lax
from jax.experimental import pallas as pl
from jax.experimental.pallas import tpu as pltpu

N_DEV = 16
N_LOC = 2
CAP = 64
SLAB = N_LOC * CAP
G = 4
W = 4


def _moe_body(send_ref, w1_ref, w2_ref, ret_ref, recv_ref,
              send_sems, recv_sems, send_sems2, recv_sems2):
    me = lax.axis_index("i")

    bsem = pltpu.get_barrier_semaphore()
    for k in range(N_DEV):
        pl.when(me != k)(
            lambda k=k: pl.semaphore_signal(
                bsem, inc=1, device_id=(k,),
                device_id_type=pl.DeviceIdType.MESH))
    pl.semaphore_wait(bsem, N_DEV - 1)

    d_model = send_ref.shape[-1]

    def disp(r):
        return pltpu.make_async_remote_copy(
            src_ref=send_ref.at[r],
            dst_ref=recv_ref.at[r],
            send_sem=send_sems.at[r],
            recv_sem=recv_sems.at[r],
            device_id=(lax.rem(me + r, N_DEV),),
            device_id_type=pl.DeviceIdType.MESH,
        )

    def comb(r):
        return pltpu.make_async_remote_copy(
            src_ref=recv_ref.at[r],
            dst_ref=ret_ref.at[r],
            send_sem=send_sems2.at[r],
            recv_sem=recv_sems2.at[r],
            device_id=(lax.rem(me + N_DEV - r, N_DEV),),
            device_id_type=pl.DeviceIdType.MESH,
        )

    recv_ref[0] = send_ref[0]
    dispatch = {r: disp(r) for r in range(1, N_DEV)}
    combine = {r: comb(r) for r in range(1, N_DEV)}

    for r in range(1, min(W, N_DEV - 1) + 1):
        dispatch[r].start()

    n_tiles = N_DEV // G
    for j in range(n_tiles):
        for r in range(j * G, (j + 1) * G):
            if r >= 1 and r + W < N_DEV:
                dispatch[r + W].start()
            if r >= 1:
                dispatch[r].wait_recv()

        lo = j * G
        for le in range(N_LOC):
            a = recv_ref[lo:lo + G, le].reshape(G * CAP, d_model)
            h = jnp.maximum(
                jnp.dot(a, w1_ref[le], preferred_element_type=jnp.float32),
                0.0)
            res = jnp.dot(h.astype(jnp.bfloat16), w2_ref[le],
                          preferred_element_type=jnp.float32)
            recv_ref[lo:lo + G, le] = res.astype(jnp.bfloat16).reshape(
                G, CAP, d_model)

        for r in range(lo, lo + G):
            if r == 0:
                ret_ref[0] = recv_ref[0]
            else:
                combine[r].start()

    for r in range(1, N_DEV):
        combine[r].wait_recv()
        combine[r].wait_send()
        dispatch[r].wait_send()


def kernel(x, assign, W1, W2):
    t_per, d_model = x.shape
    me = lax.axis_index("i")

    a = assign.astype(jnp.int32)
    tok = jnp.arange(t_per, dtype=jnp.int32)
    same = (a[:, None] == a[None, :]) & (tok[None, :] < tok[:, None])
    rank = jnp.sum(same, axis=1).astype(jnp.int32)
    owner = a // N_LOC
    rnd = jnp.remainder(owner - me, N_DEV)
    slot = rnd * SLAB + (a % N_LOC) * CAP + rank

    token_of_slot = jnp.zeros((N_DEV * SLAB,), jnp.int32).at[slot].set(tok)
    send = x[token_of_slot].astype(jnp.bfloat16).reshape(
        N_DEV, N_LOC, CAP, d_model)

    ret = pl.pallas_call(
        _moe_body,
        out_shape=jax.ShapeDtypeStruct((N_DEV, N_LOC, CAP, d_model),
                                       jnp.bfloat16),
        in_specs=[
            pl.BlockSpec(memory_space=pltpu.VMEM),
            pl.BlockSpec(memory_space=pltpu.VMEM),
            pl.BlockSpec(memory_space=pltpu.VMEM),
        ],
        out_specs=pl.BlockSpec(memory_space=pltpu.VMEM),
        scratch_shapes=[
            pltpu.VMEM((N_DEV, N_LOC, CAP, d_model), jnp.bfloat16),
            pltpu.SemaphoreType.DMA((N_DEV,)),
            pltpu.SemaphoreType.DMA((N_DEV,)),
            pltpu.SemaphoreType.DMA((N_DEV,)),
            pltpu.SemaphoreType.DMA((N_DEV,)),
        ],
        compiler_params=pltpu.CompilerParams(
            collective_id=0, vmem_limit_bytes=120 * 1024 * 1024),
    )(send, W1.astype(jnp.bfloat16), W2.astype(jnp.bfloat16))

    return ret.reshape(N_DEV * SLAB, d_model)[slot].astype(jnp.float32)
